# baseline (speedup 1.0000x reference)
import functools

import jax
import jax.numpy as jnp
from jax import lax
from jax.experimental import pallas as pl
from jax.experimental.pallas import tpu as pltpu
from jax.experimental.pallas import tpu_sc as plsc

N = 100000
NP = 100352
PK = NP // 8
E = 3200000
G = 64
HID = 32
FH = 16
NS = 16
NPAD = 102400
RPT = NPAD // NS
RPT_E = NP // NS
CZ = 392
CE = 400
EPT = E // NS
NCHUNK = EPT // CE
CD = 2000
EPW = E // (2 * NS)
NCHUNK_D = EPW // CD
BR = 448
NB8 = PK // BR
F32 = jnp.float32


def _sc_mesh():
    return plsc.VectorSubcoreMesh(core_axis_name="c", subcore_axis_name="s")


@functools.partial(
    pl.kernel,
    out_type=jax.ShapeDtypeStruct((2 * NPAD,), F32),
    mesh=_sc_mesh(),
    compiler_params=pltpu.CompilerParams(use_tc_tiling_on_sc=False),
    scratch_types=[
        pltpu.VMEM((CD,), jnp.int32),
        pltpu.VMEM((CD,), F32),
        pltpu.VMEM((RPT,), F32),
        pltpu.VMEM_SHARED((NPAD,), F32),
    ],
)
def _deg_kernel(dst_hbm, w_hbm, out_hbm, dst_v, w_v, zv, deg_sh):
    c = lax.axis_index("c")
    s = lax.axis_index("s")
    zero16 = jnp.zeros((16,), F32)

    def zfill(i, _):
        zv[pl.ds(i * 16, 16)] = zero16
        return ()

    lax.fori_loop(0, RPT // 16, zfill, ())
    rowbase = s * RPT
    pltpu.sync_copy(zv, deg_sh.at[pl.ds(rowbase, RPT)])
    plsc.subcore_barrier()

    tstart = (c * NS + s) * EPW

    def body(k, _):
        base = tstart + k * CD
        pltpu.sync_copy(dst_hbm.at[pl.ds(base, CD)], dst_v)
        pltpu.sync_copy(w_hbm.at[pl.ds(base, CD)], w_v)
        pltpu.sync_copy(w_v, deg_sh.at[dst_v], add=True)
        return ()

    lax.fori_loop(0, NCHUNK_D, body, ())
    plsc.subcore_barrier()
    pltpu.sync_copy(deg_sh.at[pl.ds(rowbase, RPT)], zv)
    pltpu.sync_copy(zv, out_hbm.at[pl.ds(c * NPAD + rowbase, RPT)])


@functools.partial(
    pl.kernel,
    out_type=(jax.ShapeDtypeStruct((NP, FH), F32),
              jax.ShapeDtypeStruct((NP, FH), F32)),
    mesh=_sc_mesh(),
    compiler_params=pltpu.CompilerParams(use_tc_tiling_on_sc=False),
    scratch_types=[
        pltpu.VMEM((CE,), jnp.int32),
        pltpu.VMEM((CE,), jnp.int32),
        pltpu.VMEM((CE,), jnp.int32),
        pltpu.VMEM((CE,), jnp.int32),
        pltpu.VMEM((CE,), F32),
        pltpu.VMEM((CE,), F32),
        pltpu.VMEM((CE, FH), F32),
        pltpu.VMEM((CE, FH), F32),
        pltpu.VMEM_SHARED((NP, FH), F32),
    ] + [pltpu.SemaphoreType.DMA] * 10,
)
def _edge_kernel(src_hbm, dst_hbm, w_hbm, ga_hbm, gb_hbm, oa_hbm, ob_hbm,
                 srcA, srcB, dstA, dstB, wA, wB, rowsA, rowsB, acc_sh,
                 lsA, ldA, lwA, lsB, ldB, lwB, sgA, sgB, ssA, ssB):
    c = lax.axis_index("c")
    s = lax.axis_index("s")
    zero16 = jnp.zeros((FH,), F32)

    def zfill(r, _):
        rowsA[r, :] = zero16
        return ()

    lax.fori_loop(0, CZ, zfill, ())
    rowbase = s * RPT_E
    zsrc = rowsA.at[pl.ds(0, CZ)]

    def zcopy(j, _):
        pltpu.sync_copy(zsrc, acc_sh.at[pl.ds(rowbase + j * CZ, CZ)])
        return ()

    lax.fori_loop(0, RPT_E // CZ, zcopy, ())
    plsc.subcore_barrier()

    t0 = s * EPT

    def gather(src_v, rows_v, sem):
        @pl.when(c == 0)
        def _():
            pltpu.async_copy(ga_hbm.at[src_v], rows_v, sem)

        @pl.when(c == 1)
        def _():
            pltpu.async_copy(gb_hbm.at[src_v], rows_v, sem)

    def gather_wait(src_v, rows_v, sem):
        @pl.when(c == 0)
        def _():
            pltpu.make_async_copy(ga_hbm.at[src_v], rows_v, sem).wait()

        @pl.when(c == 1)
        def _():
            pltpu.make_async_copy(gb_hbm.at[src_v], rows_v, sem).wait()

    pltpu.sync_copy(src_hbm.at[pl.ds(t0, CE)], srcA)
    pltpu.sync_copy(dst_hbm.at[pl.ds(t0, CE)], dstA)
    pltpu.sync_copy(w_hbm.at[pl.ds(t0, CE)], wA)
    gather(srcA, rowsA, sgA)

    def section(k, src_c, dst_c, w_c, rows_c, sg_c, ss_c,
                src_n, dst_n, w_n, rows_n, ls_n, ld_n, lw_n, sg_n, ss_n):
        @pl.when(k > 0)
        def _():
            pltpu.make_async_copy(rows_n, acc_sh.at[dst_n], ss_n).wait()

        nb = t0 + (k + 1) * CE

        @pl.when(k + 1 < NCHUNK)
        def _():
            pltpu.async_copy(src_hbm.at[pl.ds(nb, CE)], src_n, ls_n)
            pltpu.async_copy(dst_hbm.at[pl.ds(nb, CE)], dst_n, ld_n)
            pltpu.async_copy(w_hbm.at[pl.ds(nb, CE)], w_n, lw_n)

        gather_wait(src_c, rows_c, sg_c)

        @pl.when(k + 1 < NCHUNK)
        def _():
            pltpu.make_async_copy(src_hbm.at[pl.ds(nb, CE)], src_n, ls_n).wait()
            pltpu.make_async_copy(dst_hbm.at[pl.ds(nb, CE)], dst_n, ld_n).wait()
            pltpu.make_async_copy(w_hbm.at[pl.ds(nb, CE)], w_n, lw_n).wait()
            gather(src_n, rows_n, sg_n)

        zlane = lax.broadcasted_iota(jnp.int32, (16,), 0) * 0

        @plsc.parallel_loop(0, CE // 16, unroll=4)
        def _(j):
            w16 = w_c[pl.ds(j * 16, 16)]
            for t in range(16):
                r = j * 16 + t
                bc = jnp.take_along_axis(w16, zlane + t, axis=0,
                                         mode="promise_in_bounds")
                rows_c[r, :] = rows_c[r, :] * bc

        pltpu.async_copy(rows_c, acc_sh.at[dst_c], ss_c, add=True)

    def pair(p, _):
        k = 2 * p
        section(k, srcA, dstA, wA, rowsA, sgA, ssA,
                srcB, dstB, wB, rowsB, lsB, ldB, lwB, sgB, ssB)
        section(k + 1, srcB, dstB, wB, rowsB, sgB, ssB,
                srcA, dstA, wA, rowsA, lsA, ldA, lwA, sgA, ssA)
        return ()

    lax.fori_loop(0, NCHUNK // 2, pair, ())
    pltpu.make_async_copy(rowsB, acc_sh.at[dstB], ssB).wait()
    plsc.subcore_barrier()

    def ocopy(j, _):
        r0 = rowbase + j * CZ
        pltpu.sync_copy(acc_sh.at[pl.ds(r0, CZ)], zsrc)

        @pl.when(c == 0)
        def _():
            pltpu.sync_copy(zsrc, oa_hbm.at[pl.ds(r0, CZ)])

        @pl.when(c == 1)
        def _():
            pltpu.sync_copy(zsrc, ob_hbm.at[pl.ds(r0, CZ)])

        return ()

    lax.fori_loop(0, RPT_E // CZ, ocopy, ())


def _prep_body(d0, d1, x, bw1a, bw1b, rmat, dis_o, ga_o, gb_o):
    deg = d0[...] + d1[...] + 1.0
    dis8 = jnp.where(deg > 0, lax.rsqrt(deg), 0.0)
    disp = jnp.dot(dis8, rmat[...], preferred_element_type=F32)
    dis_o[...] = disp
    ga_o[...] = disp * jnp.dot(x[...], bw1a[...], preferred_element_type=F32)
    gb_o[...] = disp * jnp.dot(x[...], bw1b[...], preferred_element_type=F32)


def _mid_body(ma, mb, ga, gb, dis, b0, b1, waa, wab, wba, wbb, ga_o, gb_o):
    d = dis[...]
    h0 = jnp.maximum(d * (ma[...] + ga[...]) + b0[...], 0.0)
    h1 = jnp.maximum(d * (mb[...] + gb[...]) + b1[...], 0.0)
    hla = (jnp.dot(h0, waa[...], preferred_element_type=F32)
           + jnp.dot(h1, wab[...], preferred_element_type=F32))
    hlb = (jnp.dot(h0, wba[...], preferred_element_type=F32)
           + jnp.dot(h1, wbb[...], preferred_element_type=F32))
    ga_o[...] = d * hla
    gb_o[...] = d * hlb


def _last_body(ma, mb, ga, gb, dis, b0, b1, c0, c1, attb, batch8,
               h3a_o, h3b_o, att_o, seg_o, cnt_o, stats_o, acc):
    i = pl.program_id(0)
    d = dis[...]
    h0 = jnp.maximum(d * (ma[...] + ga[...]) + b0[...], 0.0)
    h1 = jnp.maximum(d * (mb[...] + gb[...]) + b1[...], 0.0)
    h3a_o[...] = h0
    h3b_o[...] = h1
    bt = batch8[...]
    att8 = (jnp.dot(h0, c0[...], preferred_element_type=F32)
            + jnp.dot(h1, c1[...], preferred_element_type=F32)
            + attb[0, 0])
    att8 = jnp.where(bt >= G, -1e30, att8)
    att_o[...] = att8

    seg = jnp.zeros((G, HID), F32)
    cnt = jnp.zeros((G, 1), F32)
    ones_col = jnp.ones((BR, 1), F32)
    for q in range(8):
        hq = jnp.concatenate(
            [h0[:, 16 * q:16 * q + 16], h1[:, 16 * q:16 * q + 16]], axis=1)
        oh = (bt[:, q:q + 1] ==
              lax.broadcasted_iota(jnp.int32, (BR, G), 1)).astype(F32)
        seg = seg + lax.dot_general(oh, hq, (((0,), (0,)), ((), ())),
                                    preferred_element_type=F32)
        cnt = cnt + lax.dot_general(oh, ones_col, (((0,), (0,)), ((), ())),
                                    preferred_element_type=F32)

    bm = jnp.max(att8)
    bs = jnp.sum(jnp.exp(att8 - bm))

    @pl.when(i == 0)
    def _():
        seg_o[...] = seg
        cnt_o[...] = cnt
        acc[0] = bm
        acc[1] = bs

    @pl.when(i > 0)
    def _():
        seg_o[...] += seg
        cnt_o[...] += cnt
        m_old = acc[0]
        m_new = jnp.maximum(m_old, bm)
        acc[0] = m_new
        acc[1] = acc[1] * jnp.exp(m_old - m_new) + bs * jnp.exp(bm - m_new)

    @pl.when(i == NB8 - 1)
    def _():
        stats_o[0, 0] = acc[0]
        stats_o[0, 1] = acc[1]


def _fin_body(stats, att8, h3a, h3b, rmat, attn_o, xwa_o, xwb_o):
    i = pl.program_id(0)
    m = stats[0, 0]
    s = stats[0, 1]
    a8 = jnp.exp(att8[...] - m) / s
    attn_o[...] = a8
    abc = jnp.dot(a8, rmat[...], preferred_element_type=F32)
    xwa = jnp.sum(abc * h3a[...], axis=0, keepdims=True)
    xwb = jnp.sum(abc * h3b[...], axis=0, keepdims=True)

    @pl.when(i == 0)
    def _():
        xwa_o[...] = xwa
        xwb_o[...] = xwb

    @pl.when(i > 0)
    def _():
        xwa_o[...] += xwa
        xwb_o[...] += xwb


def _head_body(seg, cnt, fcw, fcb, xwa, xwb, s0, s1, out_o, xw_o):
    pooled = seg[...] / jnp.maximum(cnt[...], 1.0)
    logits = lax.dot_general(pooled, fcw[...], (((1,), (1,)), ((), ())),
                             preferred_element_type=F32) + fcb[...]
    mx = jnp.max(logits, axis=1, keepdims=True)
    lse = mx + jnp.log(jnp.sum(jnp.exp(logits - mx), axis=1, keepdims=True))
    out_o[...] = logits - lse
    xw_o[...] = (jnp.dot(xwa[...], s0[...], preferred_element_type=F32)
                 + jnp.dot(xwb[...], s1[...], preferred_element_type=F32))


def _pspec(cols=128):
    return pl.BlockSpec((BR, cols), lambda i: (i, 0))


def _full_spec(shape):
    return pl.BlockSpec(shape, lambda i: tuple(0 for _ in shape))


def kernel(x, edge_index, edge_attr, batch, W1, b1, W2, b2, W3, b3,
           att_W, att_b, fc_W, fc_b):
    src = edge_index[0]
    dst = edge_index[1]
    w = edge_attr
    eye8 = jnp.eye(8, dtype=F32)

    def bd(m):
        return jnp.kron(eye8, m)

    bw1a = jnp.kron(eye8, W1[:FH].T)
    bw1b = jnp.kron(eye8, W1[FH:].T)
    waa2, wab2 = bd(W2[:FH, :FH].T), bd(W2[:FH, FH:].T)
    wba2, wbb2 = bd(W2[FH:, :FH].T), bd(W2[FH:, FH:].T)
    waa3, wab3 = bd(W3[:FH, :FH].T), bd(W3[:FH, FH:].T)
    wba3, wbb3 = bd(W3[FH:, :FH].T), bd(W3[FH:, FH:].T)
    rmat = jnp.kron(eye8, jnp.ones((1, FH), F32))
    c0 = jnp.kron(eye8, att_W[0, :FH].reshape(FH, 1))
    c1 = jnp.kron(eye8, att_W[0, FH:].reshape(FH, 1))
    s0 = jnp.concatenate([jnp.tile(jnp.eye(FH, dtype=F32), (8, 1)),
                          jnp.zeros((128, FH), F32)], axis=1)
    s1 = jnp.concatenate([jnp.zeros((128, FH), F32),
                          jnp.tile(jnp.eye(FH, dtype=F32), (8, 1))], axis=1)
    b0_1, b1_1 = jnp.tile(b1[:FH], 8).reshape(1, 128), jnp.tile(b1[FH:], 8).reshape(1, 128)
    b0_2, b1_2 = jnp.tile(b2[:FH], 8).reshape(1, 128), jnp.tile(b2[FH:], 8).reshape(1, 128)
    b0_3, b1_3 = jnp.tile(b3[:FH], 8).reshape(1, 128), jnp.tile(b3[FH:], 8).reshape(1, 128)
    attbr = att_b.reshape(1, 1)
    fcbr = fc_b.reshape(1, -1)

    x_p = jnp.pad(x, ((0, NP - N), (0, 0))).reshape(PK, 32)
    batch8 = jnp.pad(batch, (0, NP - N), constant_values=G).reshape(PK, 8)

    deg_flat = _deg_kernel(dst, w)
    degp = deg_flat.reshape(2, NPAD)
    d0 = degp[0, :NP].reshape(PK, 8)
    d1 = degp[1, :NP].reshape(PK, 8)

    dis, ga, gb = pl.pallas_call(
        _prep_body,
        grid=(NB8,),
        in_specs=[_pspec(8), _pspec(8), _pspec(32),
                  _full_spec((32, 128)), _full_spec((32, 128)),
                  _full_spec((8, 128))],
        out_specs=[_pspec(), _pspec(), _pspec()],
        out_shape=[jax.ShapeDtypeStruct((PK, 128), F32),
                   jax.ShapeDtypeStruct((PK, 128), F32),
                   jax.ShapeDtypeStruct((PK, 128), F32)],
    )(d0, d1, x_p, bw1a, bw1b, rmat)

    mid = pl.pallas_call(
        _mid_body,
        grid=(NB8,),
        in_specs=[_pspec(), _pspec(), _pspec(), _pspec(), _pspec(),
                  _full_spec((1, 128)), _full_spec((1, 128)),
                  _full_spec((128, 128)), _full_spec((128, 128)),
                  _full_spec((128, 128)), _full_spec((128, 128))],
        out_specs=[_pspec(), _pspec()],
        out_shape=[jax.ShapeDtypeStruct((PK, 128), F32),
                   jax.ShapeDtypeStruct((PK, 128), F32)],
    )

    def edge(ga_p, gb_p):
        oa, ob = _edge_kernel(src, dst, w,
                              ga_p.reshape(NP, FH), gb_p.reshape(NP, FH))
        return oa.reshape(PK, 128), ob.reshape(PK, 128)

    ma, mb = edge(ga, gb)
    ga, gb = mid(ma, mb, ga, gb, dis, b0_1, b1_1, waa2, wab2, wba2, wbb2)
    ma, mb = edge(ga, gb)
    ga, gb = mid(ma, mb, ga, gb, dis, b0_2, b1_2, waa3, wab3, wba3, wbb3)
    ma, mb = edge(ga, gb)

    h3a, h3b, att8, seg, cnt, stats = pl.pallas_call(
        _last_body,
        grid=(NB8,),
        in_specs=[_pspec(), _pspec(), _pspec(), _pspec(), _pspec(),
                  _full_spec((1, 128)), _full_spec((1, 128)),
                  _full_spec((128, 8)), _full_spec((128, 8)),
                  _full_spec((1, 1)), _pspec(8)],
        out_specs=[_pspec(), _pspec(), _pspec(8),
                   _full_spec((G, HID)), _full_spec((G, 1)),
                   pl.BlockSpec((1, 2), lambda i: (0, 0),
                                memory_space=pltpu.SMEM)],
        out_shape=[jax.ShapeDtypeStruct((PK, 128), F32),
                   jax.ShapeDtypeStruct((PK, 128), F32),
                   jax.ShapeDtypeStruct((PK, 8), F32),
                   jax.ShapeDtypeStruct((G, HID), F32),
                   jax.ShapeDtypeStruct((G, 1), F32),
                   jax.ShapeDtypeStruct((1, 2), F32)],
        scratch_shapes=[pltpu.SMEM((2,), F32)],
    )(ma, mb, ga, gb, dis, b0_3, b1_3, c0, c1, attbr, batch8)

    attn8, xwa, xwb = pl.pallas_call(
        _fin_body,
        grid=(NB8,),
        in_specs=[pl.BlockSpec((1, 2), lambda i: (0, 0),
                               memory_space=pltpu.SMEM),
                  _pspec(8), _pspec(), _pspec(), _full_spec((8, 128))],
        out_specs=[_pspec(8), _full_spec((1, 128)), _full_spec((1, 128))],
        out_shape=[jax.ShapeDtypeStruct((PK, 8), F32),
                   jax.ShapeDtypeStruct((1, 128), F32),
                   jax.ShapeDtypeStruct((1, 128), F32)],
    )(stats, att8, h3a, h3b, rmat)

    logw, xw = pl.pallas_call(
        _head_body,
        grid=(1,),
        in_specs=[_full_spec((G, HID)), _full_spec((G, 1)),
                  _full_spec(fc_W.shape), _full_spec((1, fc_W.shape[0])),
                  _full_spec((1, 128)), _full_spec((1, 128)),
                  _full_spec((128, HID)), _full_spec((128, HID))],
        out_specs=[_full_spec((G, fc_W.shape[0])), _full_spec((1, HID))],
        out_shape=[jax.ShapeDtypeStruct((G, fc_W.shape[0]), F32),
                   jax.ShapeDtypeStruct((1, HID), F32)],
    )(seg, cnt, fc_W, fcbr, xwa, xwb, s0, s1)

    attn = attn8.reshape(NP, 1)[:N]
    return (logw, attn, xw.reshape(HID))


# speedup vs baseline: 1.2064x; 1.2064x over previous
import functools

import jax
import jax.numpy as jnp
from jax import lax
from jax.experimental import pallas as pl
from jax.experimental.pallas import tpu as pltpu
from jax.experimental.pallas import tpu_sc as plsc

N = 100000
NP = 100352
PK = NP // 8
E = 3200000
G = 64
HID = 32
FH = 16
NS = 16
NPAD = 102400
RPT = NPAD // NS
RPT_E = NP // NS
CZ = 392
CE = 400
EPT = E // NS
NCHUNK = EPT // CE
CD = 2000
EPW = E // (2 * NS)
NCHUNK_D = EPW // CD
BR = 448
NB8 = PK // BR
F32 = jnp.float32


def _sc_mesh():
    return plsc.VectorSubcoreMesh(core_axis_name="c", subcore_axis_name="s")


@functools.partial(
    pl.kernel,
    out_type=jax.ShapeDtypeStruct((2 * NPAD,), F32),
    mesh=_sc_mesh(),
    compiler_params=pltpu.CompilerParams(use_tc_tiling_on_sc=False),
    scratch_types=[
        pltpu.VMEM((CD,), jnp.int32),
        pltpu.VMEM((CD,), F32),
        pltpu.VMEM((RPT,), F32),
        pltpu.VMEM_SHARED((NPAD,), F32),
    ],
)
def _deg_kernel(dst_hbm, w_hbm, out_hbm, dst_v, w_v, zv, deg_sh):
    c = lax.axis_index("c")
    s = lax.axis_index("s")
    zero16 = jnp.zeros((16,), F32)

    def zfill(i, _):
        zv[pl.ds(i * 16, 16)] = zero16
        return ()

    lax.fori_loop(0, RPT // 16, zfill, ())
    rowbase = s * RPT
    pltpu.sync_copy(zv, deg_sh.at[pl.ds(rowbase, RPT)])
    plsc.subcore_barrier()

    tstart = (c * NS + s) * EPW

    def body(k, _):
        base = tstart + k * CD
        pltpu.sync_copy(dst_hbm.at[pl.ds(base, CD)], dst_v)
        pltpu.sync_copy(w_hbm.at[pl.ds(base, CD)], w_v)
        pltpu.sync_copy(w_v, deg_sh.at[dst_v], add=True)
        return ()

    lax.fori_loop(0, NCHUNK_D, body, ())
    plsc.subcore_barrier()
    pltpu.sync_copy(deg_sh.at[pl.ds(rowbase, RPT)], zv)
    pltpu.sync_copy(zv, out_hbm.at[pl.ds(c * NPAD + rowbase, RPT)])


@functools.partial(
    pl.kernel,
    out_type=(jax.ShapeDtypeStruct((NP, FH), F32),
              jax.ShapeDtypeStruct((NP, FH), F32)),
    mesh=_sc_mesh(),
    compiler_params=pltpu.CompilerParams(use_tc_tiling_on_sc=False),
    scratch_types=[
        pltpu.VMEM((CE,), jnp.int32),
        pltpu.VMEM((CE,), jnp.int32),
        pltpu.VMEM((CE,), jnp.int32),
        pltpu.VMEM((CE,), jnp.int32),
        pltpu.VMEM((CE,), F32),
        pltpu.VMEM((CE,), F32),
        pltpu.VMEM((CE, FH), F32),
        pltpu.VMEM((CE, FH), F32),
        pltpu.VMEM_SHARED((NP, FH), F32),
    ] + [pltpu.SemaphoreType.DMA] * 10,
)
def _edge_kernel(src_hbm, dst_hbm, w_hbm, ga_hbm, gb_hbm, oa_hbm, ob_hbm,
                 srcA, srcB, dstA, dstB, wA, wB, rowsA, rowsB, acc_sh,
                 lsA, ldA, lwA, lsB, ldB, lwB, sgA, sgB, ssA, ssB):
    c = lax.axis_index("c")
    s = lax.axis_index("s")
    zero16 = jnp.zeros((FH,), F32)

    def zfill(r, _):
        rowsA[r, :] = zero16
        return ()

    lax.fori_loop(0, CZ, zfill, ())
    rowbase = s * RPT_E
    zsrc = rowsA.at[pl.ds(0, CZ)]

    def zcopy(j, _):
        pltpu.sync_copy(zsrc, acc_sh.at[pl.ds(rowbase + j * CZ, CZ)])
        return ()

    lax.fori_loop(0, RPT_E // CZ, zcopy, ())
    plsc.subcore_barrier()

    t0 = s * EPT

    def gather(src_v, rows_v, sem):
        @pl.when(c == 0)
        def _():
            pltpu.async_copy(ga_hbm.at[src_v], rows_v, sem)

        @pl.when(c == 1)
        def _():
            pltpu.async_copy(gb_hbm.at[src_v], rows_v, sem)

    def gather_wait(src_v, rows_v, sem):
        @pl.when(c == 0)
        def _():
            pltpu.make_async_copy(ga_hbm.at[src_v], rows_v, sem).wait()

        @pl.when(c == 1)
        def _():
            pltpu.make_async_copy(gb_hbm.at[src_v], rows_v, sem).wait()

    pltpu.sync_copy(src_hbm.at[pl.ds(t0, CE)], srcA)
    pltpu.sync_copy(dst_hbm.at[pl.ds(t0, CE)], dstA)
    pltpu.sync_copy(w_hbm.at[pl.ds(t0, CE)], wA)
    gather(srcA, rowsA, sgA)

    def section(k, src_c, dst_c, w_c, rows_c, sg_c, ss_c,
                src_n, dst_n, w_n, rows_n, ls_n, ld_n, lw_n, sg_n, ss_n):
        @pl.when(k > 0)
        def _():
            pltpu.make_async_copy(rows_n, acc_sh.at[dst_n], ss_n).wait()

        nb = t0 + (k + 1) * CE

        @pl.when(k + 1 < NCHUNK)
        def _():
            pltpu.async_copy(src_hbm.at[pl.ds(nb, CE)], src_n, ls_n)
            pltpu.async_copy(dst_hbm.at[pl.ds(nb, CE)], dst_n, ld_n)
            pltpu.async_copy(w_hbm.at[pl.ds(nb, CE)], w_n, lw_n)

        gather_wait(src_c, rows_c, sg_c)

        @pl.when(k + 1 < NCHUNK)
        def _():
            pltpu.make_async_copy(src_hbm.at[pl.ds(nb, CE)], src_n, ls_n).wait()
            pltpu.make_async_copy(dst_hbm.at[pl.ds(nb, CE)], dst_n, ld_n).wait()
            pltpu.make_async_copy(w_hbm.at[pl.ds(nb, CE)], w_n, lw_n).wait()
            gather(src_n, rows_n, sg_n)

        zlane = lax.broadcasted_iota(jnp.int32, (16,), 0) * 0

        @plsc.parallel_loop(0, CE // 16, unroll=1)
        def _(j):
            w16 = w_c[pl.ds(j * 16, 16)]
            for t in range(16):
                r = j * 16 + t
                bc = jnp.take_along_axis(w16, zlane + t, axis=0,
                                         mode="promise_in_bounds")
                rows_c[r, :] = rows_c[r, :] * bc

        pltpu.async_copy(rows_c, acc_sh.at[dst_c], ss_c, add=True)

    def pair(p, _):
        k = 2 * p
        section(k, srcA, dstA, wA, rowsA, sgA, ssA,
                srcB, dstB, wB, rowsB, lsB, ldB, lwB, sgB, ssB)
        section(k + 1, srcB, dstB, wB, rowsB, sgB, ssB,
                srcA, dstA, wA, rowsA, lsA, ldA, lwA, sgA, ssA)
        return ()

    lax.fori_loop(0, NCHUNK // 2, pair, ())
    pltpu.make_async_copy(rowsB, acc_sh.at[dstB], ssB).wait()
    plsc.subcore_barrier()

    def ocopy(j, _):
        r0 = rowbase + j * CZ
        pltpu.sync_copy(acc_sh.at[pl.ds(r0, CZ)], zsrc)

        @pl.when(c == 0)
        def _():
            pltpu.sync_copy(zsrc, oa_hbm.at[pl.ds(r0, CZ)])

        @pl.when(c == 1)
        def _():
            pltpu.sync_copy(zsrc, ob_hbm.at[pl.ds(r0, CZ)])

        return ()

    lax.fori_loop(0, RPT_E // CZ, ocopy, ())


def _prep_body(d0, d1, x, bw1a, bw1b, rmat, dis_o, ga_o, gb_o):
    deg = d0[...] + d1[...] + 1.0
    dis8 = jnp.where(deg > 0, lax.rsqrt(deg), 0.0)
    disp = jnp.dot(dis8, rmat[...], preferred_element_type=F32)
    dis_o[...] = disp
    ga_o[...] = disp * jnp.dot(x[...], bw1a[...], preferred_element_type=F32)
    gb_o[...] = disp * jnp.dot(x[...], bw1b[...], preferred_element_type=F32)


def _mid_body(ma, mb, ga, gb, dis, b0, b1, waa, wab, wba, wbb, ga_o, gb_o):
    d = dis[...]
    h0 = jnp.maximum(d * (ma[...] + ga[...]) + b0[...], 0.0)
    h1 = jnp.maximum(d * (mb[...] + gb[...]) + b1[...], 0.0)
    hla = (jnp.dot(h0, waa[...], preferred_element_type=F32)
           + jnp.dot(h1, wab[...], preferred_element_type=F32))
    hlb = (jnp.dot(h0, wba[...], preferred_element_type=F32)
           + jnp.dot(h1, wbb[...], preferred_element_type=F32))
    ga_o[...] = d * hla
    gb_o[...] = d * hlb


def _last_body(ma, mb, ga, gb, dis, b0, b1, c0, c1, attb, batch8,
               h3a_o, h3b_o, att_o, seg_o, cnt_o, stats_o, acc):
    i = pl.program_id(0)
    d = dis[...]
    h0 = jnp.maximum(d * (ma[...] + ga[...]) + b0[...], 0.0)
    h1 = jnp.maximum(d * (mb[...] + gb[...]) + b1[...], 0.0)
    h3a_o[...] = h0
    h3b_o[...] = h1
    bt = batch8[...]
    att8 = (jnp.dot(h0, c0[...], preferred_element_type=F32)
            + jnp.dot(h1, c1[...], preferred_element_type=F32)
            + attb[0, 0])
    att8 = jnp.where(bt >= G, -1e30, att8)
    att_o[...] = att8

    seg = jnp.zeros((G, HID), F32)
    cnt = jnp.zeros((G, 1), F32)
    ones_col = jnp.ones((BR, 1), F32)
    for q in range(8):
        hq = jnp.concatenate(
            [h0[:, 16 * q:16 * q + 16], h1[:, 16 * q:16 * q + 16]], axis=1)
        oh = (bt[:, q:q + 1] ==
              lax.broadcasted_iota(jnp.int32, (BR, G), 1)).astype(F32)
        seg = seg + lax.dot_general(oh, hq, (((0,), (0,)), ((), ())),
                                    preferred_element_type=F32)
        cnt = cnt + lax.dot_general(oh, ones_col, (((0,), (0,)), ((), ())),
                                    preferred_element_type=F32)

    bm = jnp.max(att8)
    bs = jnp.sum(jnp.exp(att8 - bm))

    @pl.when(i == 0)
    def _():
        seg_o[...] = seg
        cnt_o[...] = cnt
        acc[0] = bm
        acc[1] = bs

    @pl.when(i > 0)
    def _():
        seg_o[...] += seg
        cnt_o[...] += cnt
        m_old = acc[0]
        m_new = jnp.maximum(m_old, bm)
        acc[0] = m_new
        acc[1] = acc[1] * jnp.exp(m_old - m_new) + bs * jnp.exp(bm - m_new)

    @pl.when(i == NB8 - 1)
    def _():
        stats_o[0, 0] = acc[0]
        stats_o[0, 1] = acc[1]


def _fin_body(stats, att8, h3a, h3b, rmat, attn_o, xwa_o, xwb_o):
    i = pl.program_id(0)
    m = stats[0, 0]
    s = stats[0, 1]
    a8 = jnp.exp(att8[...] - m) / s
    attn_o[...] = a8
    abc = jnp.dot(a8, rmat[...], preferred_element_type=F32)
    xwa = jnp.sum(abc * h3a[...], axis=0, keepdims=True)
    xwb = jnp.sum(abc * h3b[...], axis=0, keepdims=True)

    @pl.when(i == 0)
    def _():
        xwa_o[...] = xwa
        xwb_o[...] = xwb

    @pl.when(i > 0)
    def _():
        xwa_o[...] += xwa
        xwb_o[...] += xwb


def _head_body(seg, cnt, fcw, fcb, xwa, xwb, s0, s1, out_o, xw_o):
    pooled = seg[...] / jnp.maximum(cnt[...], 1.0)
    logits = lax.dot_general(pooled, fcw[...], (((1,), (1,)), ((), ())),
                             preferred_element_type=F32) + fcb[...]
    mx = jnp.max(logits, axis=1, keepdims=True)
    lse = mx + jnp.log(jnp.sum(jnp.exp(logits - mx), axis=1, keepdims=True))
    out_o[...] = logits - lse
    xw_o[...] = (jnp.dot(xwa[...], s0[...], preferred_element_type=F32)
                 + jnp.dot(xwb[...], s1[...], preferred_element_type=F32))


def _pspec(cols=128):
    return pl.BlockSpec((BR, cols), lambda i: (i, 0))


def _full_spec(shape):
    return pl.BlockSpec(shape, lambda i: tuple(0 for _ in shape))


def kernel(x, edge_index, edge_attr, batch, W1, b1, W2, b2, W3, b3,
           att_W, att_b, fc_W, fc_b):
    src = edge_index[0]
    dst = edge_index[1]
    w = edge_attr
    eye8 = jnp.eye(8, dtype=F32)

    def bd(m):
        return jnp.kron(eye8, m)

    bw1a = jnp.kron(eye8, W1[:FH].T)
    bw1b = jnp.kron(eye8, W1[FH:].T)
    waa2, wab2 = bd(W2[:FH, :FH].T), bd(W2[:FH, FH:].T)
    wba2, wbb2 = bd(W2[FH:, :FH].T), bd(W2[FH:, FH:].T)
    waa3, wab3 = bd(W3[:FH, :FH].T), bd(W3[:FH, FH:].T)
    wba3, wbb3 = bd(W3[FH:, :FH].T), bd(W3[FH:, FH:].T)
    rmat = jnp.kron(eye8, jnp.ones((1, FH), F32))
    c0 = jnp.kron(eye8, att_W[0, :FH].reshape(FH, 1))
    c1 = jnp.kron(eye8, att_W[0, FH:].reshape(FH, 1))
    s0 = jnp.concatenate([jnp.tile(jnp.eye(FH, dtype=F32), (8, 1)),
                          jnp.zeros((128, FH), F32)], axis=1)
    s1 = jnp.concatenate([jnp.zeros((128, FH), F32),
                          jnp.tile(jnp.eye(FH, dtype=F32), (8, 1))], axis=1)
    b0_1, b1_1 = jnp.tile(b1[:FH], 8).reshape(1, 128), jnp.tile(b1[FH:], 8).reshape(1, 128)
    b0_2, b1_2 = jnp.tile(b2[:FH], 8).reshape(1, 128), jnp.tile(b2[FH:], 8).reshape(1, 128)
    b0_3, b1_3 = jnp.tile(b3[:FH], 8).reshape(1, 128), jnp.tile(b3[FH:], 8).reshape(1, 128)
    attbr = att_b.reshape(1, 1)
    fcbr = fc_b.reshape(1, -1)

    x_p = jnp.pad(x, ((0, NP - N), (0, 0))).reshape(PK, 32)
    batch8 = jnp.pad(batch, (0, NP - N), constant_values=G).reshape(PK, 8)

    deg_flat = _deg_kernel(dst, w)
    degp = deg_flat.reshape(2, NPAD)
    d0 = degp[0, :NP].reshape(PK, 8)
    d1 = degp[1, :NP].reshape(PK, 8)

    dis, ga, gb = pl.pallas_call(
        _prep_body,
        grid=(NB8,),
        in_specs=[_pspec(8), _pspec(8), _pspec(32),
                  _full_spec((32, 128)), _full_spec((32, 128)),
                  _full_spec((8, 128))],
        out_specs=[_pspec(), _pspec(), _pspec()],
        out_shape=[jax.ShapeDtypeStruct((PK, 128), F32),
                   jax.ShapeDtypeStruct((PK, 128), F32),
                   jax.ShapeDtypeStruct((PK, 128), F32)],
    )(d0, d1, x_p, bw1a, bw1b, rmat)

    mid = pl.pallas_call(
        _mid_body,
        grid=(NB8,),
        in_specs=[_pspec(), _pspec(), _pspec(), _pspec(), _pspec(),
                  _full_spec((1, 128)), _full_spec((1, 128)),
                  _full_spec((128, 128)), _full_spec((128, 128)),
                  _full_spec((128, 128)), _full_spec((128, 128))],
        out_specs=[_pspec(), _pspec()],
        out_shape=[jax.ShapeDtypeStruct((PK, 128), F32),
                   jax.ShapeDtypeStruct((PK, 128), F32)],
    )

    def edge(ga_p, gb_p):
        oa, ob = _edge_kernel(src, dst, w,
                              ga_p.reshape(NP, FH), gb_p.reshape(NP, FH))
        return oa.reshape(PK, 128), ob.reshape(PK, 128)

    ma, mb = edge(ga, gb)
    ga, gb = mid(ma, mb, ga, gb, dis, b0_1, b1_1, waa2, wab2, wba2, wbb2)
    ma, mb = edge(ga, gb)
    ga, gb = mid(ma, mb, ga, gb, dis, b0_2, b1_2, waa3, wab3, wba3, wbb3)
    ma, mb = edge(ga, gb)

    h3a, h3b, att8, seg, cnt, stats = pl.pallas_call(
        _last_body,
        grid=(NB8,),
        in_specs=[_pspec(), _pspec(), _pspec(), _pspec(), _pspec(),
                  _full_spec((1, 128)), _full_spec((1, 128)),
                  _full_spec((128, 8)), _full_spec((128, 8)),
                  _full_spec((1, 1)), _pspec(8)],
        out_specs=[_pspec(), _pspec(), _pspec(8),
                   _full_spec((G, HID)), _full_spec((G, 1)),
                   pl.BlockSpec((1, 2), lambda i: (0, 0),
                                memory_space=pltpu.SMEM)],
        out_shape=[jax.ShapeDtypeStruct((PK, 128), F32),
                   jax.ShapeDtypeStruct((PK, 128), F32),
                   jax.ShapeDtypeStruct((PK, 8), F32),
                   jax.ShapeDtypeStruct((G, HID), F32),
                   jax.ShapeDtypeStruct((G, 1), F32),
                   jax.ShapeDtypeStruct((1, 2), F32)],
        scratch_shapes=[pltpu.SMEM((2,), F32)],
    )(ma, mb, ga, gb, dis, b0_3, b1_3, c0, c1, attbr, batch8)

    attn8, xwa, xwb = pl.pallas_call(
        _fin_body,
        grid=(NB8,),
        in_specs=[pl.BlockSpec((1, 2), lambda i: (0, 0),
                               memory_space=pltpu.SMEM),
                  _pspec(8), _pspec(), _pspec(), _full_spec((8, 128))],
        out_specs=[_pspec(8), _full_spec((1, 128)), _full_spec((1, 128))],
        out_shape=[jax.ShapeDtypeStruct((PK, 8), F32),
                   jax.ShapeDtypeStruct((1, 128), F32),
                   jax.ShapeDtypeStruct((1, 128), F32)],
    )(stats, att8, h3a, h3b, rmat)

    logw, xw = pl.pallas_call(
        _head_body,
        grid=(1,),
        in_specs=[_full_spec((G, HID)), _full_spec((G, 1)),
                  _full_spec(fc_W.shape), _full_spec((1, fc_W.shape[0])),
                  _full_spec((1, 128)), _full_spec((1, 128)),
                  _full_spec((128, HID)), _full_spec((128, HID))],
        out_specs=[_full_spec((G, fc_W.shape[0])), _full_spec((1, HID))],
        out_shape=[jax.ShapeDtypeStruct((G, fc_W.shape[0]), F32),
                   jax.ShapeDtypeStruct((1, HID), F32)],
    )(seg, cnt, fc_W, fcbr, xwa, xwb, s0, s1)

    attn = attn8.reshape(NP, 1)[:N]
    return (logw, attn, xw.reshape(HID))


# speedup vs baseline: 1.2127x; 1.0052x over previous
import functools

import jax
import jax.numpy as jnp
from jax import lax
from jax.experimental import pallas as pl
from jax.experimental.pallas import tpu as pltpu
from jax.experimental.pallas import tpu_sc as plsc

N = 100000
NP = 100352
PK = NP // 8
E = 3200000
G = 64
HID = 32
FH = 16
NS = 16
NPAD = 102400
RPT = NPAD // NS
RPT_E = NP // NS
CZ = 392
CE = 400
EPT = E // NS
NCHUNK = EPT // CE
CD = 2000
EPW = E // (2 * NS)
NCHUNK_D = EPW // CD
BR = 448
NB8 = PK // BR
F32 = jnp.float32


def _sc_mesh():
    return plsc.VectorSubcoreMesh(core_axis_name="c", subcore_axis_name="s")


@functools.partial(
    pl.kernel,
    out_type=jax.ShapeDtypeStruct((2 * NPAD,), F32),
    mesh=_sc_mesh(),
    compiler_params=pltpu.CompilerParams(use_tc_tiling_on_sc=False),
    scratch_types=[
        pltpu.VMEM((CD,), jnp.int32),
        pltpu.VMEM((CD,), F32),
        pltpu.VMEM((RPT,), F32),
        pltpu.VMEM_SHARED((NPAD,), F32),
    ],
)
def _deg_kernel(dst_hbm, w_hbm, out_hbm, dst_v, w_v, zv, deg_sh):
    c = lax.axis_index("c")
    s = lax.axis_index("s")
    zero16 = jnp.zeros((16,), F32)

    def zfill(i, _):
        zv[pl.ds(i * 16, 16)] = zero16
        return ()

    lax.fori_loop(0, RPT // 16, zfill, ())
    rowbase = s * RPT
    pltpu.sync_copy(zv, deg_sh.at[pl.ds(rowbase, RPT)])
    plsc.subcore_barrier()

    tstart = (c * NS + s) * EPW

    def body(k, _):
        base = tstart + k * CD
        pltpu.sync_copy(dst_hbm.at[pl.ds(base, CD)], dst_v)
        pltpu.sync_copy(w_hbm.at[pl.ds(base, CD)], w_v)
        pltpu.sync_copy(w_v, deg_sh.at[dst_v], add=True)
        return ()

    lax.fori_loop(0, NCHUNK_D, body, ())
    plsc.subcore_barrier()
    pltpu.sync_copy(deg_sh.at[pl.ds(rowbase, RPT)], zv)
    pltpu.sync_copy(zv, out_hbm.at[pl.ds(c * NPAD + rowbase, RPT)])


@functools.partial(
    pl.kernel,
    out_type=(jax.ShapeDtypeStruct((NP, FH), F32),
              jax.ShapeDtypeStruct((NP, FH), F32)),
    mesh=_sc_mesh(),
    compiler_params=pltpu.CompilerParams(use_tc_tiling_on_sc=False),
    scratch_types=[
        pltpu.VMEM((CE,), jnp.int32),
        pltpu.VMEM((CE,), jnp.int32),
        pltpu.VMEM((CE,), jnp.int32),
        pltpu.VMEM((CE,), jnp.int32),
        pltpu.VMEM((CE,), F32),
        pltpu.VMEM((CE,), F32),
        pltpu.VMEM((CE, FH), F32),
        pltpu.VMEM((CE, FH), F32),
        pltpu.VMEM_SHARED((NP, FH), F32),
    ] + [pltpu.SemaphoreType.DMA] * 12,
)
def _edge_kernel(src_hbm, dst_hbm, w_hbm, ga_hbm, gb_hbm, oa_hbm, ob_hbm,
                 srcA, srcB, dstA, dstB, wA, wB, rowsA, rowsB, acc_sh,
                 lsA, ldA, lwA, lsB, ldB, lwB, sgA, sgB, ssA, ssB,
                 sg2A, sg2B):
    c = lax.axis_index("c")
    s = lax.axis_index("s")
    zero16 = jnp.zeros((FH,), F32)

    def zfill(r, _):
        rowsA[r, :] = zero16
        return ()

    lax.fori_loop(0, CZ, zfill, ())
    rowbase = s * RPT_E
    zsrc = rowsA.at[pl.ds(0, CZ)]

    def zcopy(j, _):
        pltpu.sync_copy(zsrc, acc_sh.at[pl.ds(rowbase + j * CZ, CZ)])
        return ()

    lax.fori_loop(0, RPT_E // CZ, zcopy, ())
    plsc.subcore_barrier()

    t0 = s * EPT

    CH = CE // 2

    def _gparts(src_v, rows_v):
        return ((src_v.at[pl.ds(0, CH)], rows_v.at[pl.ds(0, CH)]),
                (src_v.at[pl.ds(CH, CH)], rows_v.at[pl.ds(CH, CH)]))

    def gather(src_v, rows_v, sem, sem2):
        (i1, r1), (i2, r2) = _gparts(src_v, rows_v)

        @pl.when(c == 0)
        def _():
            pltpu.async_copy(ga_hbm.at[i1], r1, sem)
            pltpu.async_copy(ga_hbm.at[i2], r2, sem2)

        @pl.when(c == 1)
        def _():
            pltpu.async_copy(gb_hbm.at[i1], r1, sem)
            pltpu.async_copy(gb_hbm.at[i2], r2, sem2)

    def gather_wait(src_v, rows_v, sem, sem2):
        (i1, r1), (i2, r2) = _gparts(src_v, rows_v)

        @pl.when(c == 0)
        def _():
            pltpu.make_async_copy(ga_hbm.at[i1], r1, sem).wait()
            pltpu.make_async_copy(ga_hbm.at[i2], r2, sem2).wait()

        @pl.when(c == 1)
        def _():
            pltpu.make_async_copy(gb_hbm.at[i1], r1, sem).wait()
            pltpu.make_async_copy(gb_hbm.at[i2], r2, sem2).wait()

    pltpu.sync_copy(src_hbm.at[pl.ds(t0, CE)], srcA)
    pltpu.sync_copy(dst_hbm.at[pl.ds(t0, CE)], dstA)
    pltpu.sync_copy(w_hbm.at[pl.ds(t0, CE)], wA)
    gather(srcA, rowsA, sgA, sg2A)

    def section(k, src_c, dst_c, w_c, rows_c, sg_c, sg2_c, ss_c,
                src_n, dst_n, w_n, rows_n, ls_n, ld_n, lw_n, sg_n, sg2_n,
                ss_n):
        @pl.when(k > 0)
        def _():
            pltpu.make_async_copy(rows_n, acc_sh.at[dst_n], ss_n).wait()

        nb = t0 + (k + 1) * CE

        @pl.when(k + 1 < NCHUNK)
        def _():
            pltpu.async_copy(src_hbm.at[pl.ds(nb, CE)], src_n, ls_n)
            pltpu.async_copy(dst_hbm.at[pl.ds(nb, CE)], dst_n, ld_n)
            pltpu.async_copy(w_hbm.at[pl.ds(nb, CE)], w_n, lw_n)

        gather_wait(src_c, rows_c, sg_c, sg2_c)

        @pl.when(k + 1 < NCHUNK)
        def _():
            pltpu.make_async_copy(src_hbm.at[pl.ds(nb, CE)], src_n, ls_n).wait()
            pltpu.make_async_copy(dst_hbm.at[pl.ds(nb, CE)], dst_n, ld_n).wait()
            pltpu.make_async_copy(w_hbm.at[pl.ds(nb, CE)], w_n, lw_n).wait()
            gather(src_n, rows_n, sg_n, sg2_n)

        zlane = lax.broadcasted_iota(jnp.int32, (16,), 0) * 0

        @plsc.parallel_loop(0, CE // 16, unroll=1)
        def _(j):
            w16 = w_c[pl.ds(j * 16, 16)]
            for t in range(16):
                r = j * 16 + t
                bc = jnp.take_along_axis(w16, zlane + t, axis=0,
                                         mode="promise_in_bounds")
                rows_c[r, :] = rows_c[r, :] * bc

        pltpu.async_copy(rows_c, acc_sh.at[dst_c], ss_c, add=True)

    def pair(p, _):
        k = 2 * p
        section(k, srcA, dstA, wA, rowsA, sgA, sg2A, ssA,
                srcB, dstB, wB, rowsB, lsB, ldB, lwB, sgB, sg2B, ssB)
        section(k + 1, srcB, dstB, wB, rowsB, sgB, sg2B, ssB,
                srcA, dstA, wA, rowsA, lsA, ldA, lwA, sgA, sg2A, ssA)
        return ()

    lax.fori_loop(0, NCHUNK // 2, pair, ())
    pltpu.make_async_copy(rowsB, acc_sh.at[dstB], ssB).wait()
    plsc.subcore_barrier()

    def ocopy(j, _):
        r0 = rowbase + j * CZ
        pltpu.sync_copy(acc_sh.at[pl.ds(r0, CZ)], zsrc)

        @pl.when(c == 0)
        def _():
            pltpu.sync_copy(zsrc, oa_hbm.at[pl.ds(r0, CZ)])

        @pl.when(c == 1)
        def _():
            pltpu.sync_copy(zsrc, ob_hbm.at[pl.ds(r0, CZ)])

        return ()

    lax.fori_loop(0, RPT_E // CZ, ocopy, ())


def _prep_body(d0, d1, x, bw1a, bw1b, rmat, dis_o, ga_o, gb_o):
    deg = d0[...] + d1[...] + 1.0
    dis8 = jnp.where(deg > 0, lax.rsqrt(deg), 0.0)
    disp = jnp.dot(dis8, rmat[...], preferred_element_type=F32)
    dis_o[...] = disp
    ga_o[...] = disp * jnp.dot(x[...], bw1a[...], preferred_element_type=F32)
    gb_o[...] = disp * jnp.dot(x[...], bw1b[...], preferred_element_type=F32)


def _mid_body(ma, mb, ga, gb, dis, b0, b1, waa, wab, wba, wbb, ga_o, gb_o):
    d = dis[...]
    h0 = jnp.maximum(d * (ma[...] + ga[...]) + b0[...], 0.0)
    h1 = jnp.maximum(d * (mb[...] + gb[...]) + b1[...], 0.0)
    hla = (jnp.dot(h0, waa[...], preferred_element_type=F32)
           + jnp.dot(h1, wab[...], preferred_element_type=F32))
    hlb = (jnp.dot(h0, wba[...], preferred_element_type=F32)
           + jnp.dot(h1, wbb[...], preferred_element_type=F32))
    ga_o[...] = d * hla
    gb_o[...] = d * hlb


def _last_body(ma, mb, ga, gb, dis, b0, b1, c0, c1, attb, batch8,
               h3a_o, h3b_o, att_o, seg_o, cnt_o, stats_o, acc):
    i = pl.program_id(0)
    d = dis[...]
    h0 = jnp.maximum(d * (ma[...] + ga[...]) + b0[...], 0.0)
    h1 = jnp.maximum(d * (mb[...] + gb[...]) + b1[...], 0.0)
    h3a_o[...] = h0
    h3b_o[...] = h1
    bt = batch8[...]
    att8 = (jnp.dot(h0, c0[...], preferred_element_type=F32)
            + jnp.dot(h1, c1[...], preferred_element_type=F32)
            + attb[0, 0])
    att8 = jnp.where(bt >= G, -1e30, att8)
    att_o[...] = att8

    seg = jnp.zeros((G, HID), F32)
    cnt = jnp.zeros((G, 1), F32)
    ones_col = jnp.ones((BR, 1), F32)
    for q in range(8):
        hq = jnp.concatenate(
            [h0[:, 16 * q:16 * q + 16], h1[:, 16 * q:16 * q + 16]], axis=1)
        oh = (bt[:, q:q + 1] ==
              lax.broadcasted_iota(jnp.int32, (BR, G), 1)).astype(F32)
        seg = seg + lax.dot_general(oh, hq, (((0,), (0,)), ((), ())),
                                    preferred_element_type=F32)
        cnt = cnt + lax.dot_general(oh, ones_col, (((0,), (0,)), ((), ())),
                                    preferred_element_type=F32)

    bm = jnp.max(att8)
    bs = jnp.sum(jnp.exp(att8 - bm))

    @pl.when(i == 0)
    def _():
        seg_o[...] = seg
        cnt_o[...] = cnt
        acc[0] = bm
        acc[1] = bs

    @pl.when(i > 0)
    def _():
        seg_o[...] += seg
        cnt_o[...] += cnt
        m_old = acc[0]
        m_new = jnp.maximum(m_old, bm)
        acc[0] = m_new
        acc[1] = acc[1] * jnp.exp(m_old - m_new) + bs * jnp.exp(bm - m_new)

    @pl.when(i == NB8 - 1)
    def _():
        stats_o[0, 0] = acc[0]
        stats_o[0, 1] = acc[1]


def _fin_body(stats, att8, h3a, h3b, rmat, attn_o, xwa_o, xwb_o):
    i = pl.program_id(0)
    m = stats[0, 0]
    s = stats[0, 1]
    a8 = jnp.exp(att8[...] - m) / s
    attn_o[...] = a8
    abc = jnp.dot(a8, rmat[...], preferred_element_type=F32)
    xwa = jnp.sum(abc * h3a[...], axis=0, keepdims=True)
    xwb = jnp.sum(abc * h3b[...], axis=0, keepdims=True)

    @pl.when(i == 0)
    def _():
        xwa_o[...] = xwa
        xwb_o[...] = xwb

    @pl.when(i > 0)
    def _():
        xwa_o[...] += xwa
        xwb_o[...] += xwb


def _head_body(seg, cnt, fcw, fcb, xwa, xwb, s0, s1, out_o, xw_o):
    pooled = seg[...] / jnp.maximum(cnt[...], 1.0)
    logits = lax.dot_general(pooled, fcw[...], (((1,), (1,)), ((), ())),
                             preferred_element_type=F32) + fcb[...]
    mx = jnp.max(logits, axis=1, keepdims=True)
    lse = mx + jnp.log(jnp.sum(jnp.exp(logits - mx), axis=1, keepdims=True))
    out_o[...] = logits - lse
    xw_o[...] = (jnp.dot(xwa[...], s0[...], preferred_element_type=F32)
                 + jnp.dot(xwb[...], s1[...], preferred_element_type=F32))


def _pspec(cols=128):
    return pl.BlockSpec((BR, cols), lambda i: (i, 0))


def _full_spec(shape):
    return pl.BlockSpec(shape, lambda i: tuple(0 for _ in shape))


def kernel(x, edge_index, edge_attr, batch, W1, b1, W2, b2, W3, b3,
           att_W, att_b, fc_W, fc_b):
    src = edge_index[0]
    dst = edge_index[1]
    w = edge_attr
    eye8 = jnp.eye(8, dtype=F32)

    def bd(m):
        return jnp.kron(eye8, m)

    bw1a = jnp.kron(eye8, W1[:FH].T)
    bw1b = jnp.kron(eye8, W1[FH:].T)
    waa2, wab2 = bd(W2[:FH, :FH].T), bd(W2[:FH, FH:].T)
    wba2, wbb2 = bd(W2[FH:, :FH].T), bd(W2[FH:, FH:].T)
    waa3, wab3 = bd(W3[:FH, :FH].T), bd(W3[:FH, FH:].T)
    wba3, wbb3 = bd(W3[FH:, :FH].T), bd(W3[FH:, FH:].T)
    rmat = jnp.kron(eye8, jnp.ones((1, FH), F32))
    c0 = jnp.kron(eye8, att_W[0, :FH].reshape(FH, 1))
    c1 = jnp.kron(eye8, att_W[0, FH:].reshape(FH, 1))
    s0 = jnp.concatenate([jnp.tile(jnp.eye(FH, dtype=F32), (8, 1)),
                          jnp.zeros((128, FH), F32)], axis=1)
    s1 = jnp.concatenate([jnp.zeros((128, FH), F32),
                          jnp.tile(jnp.eye(FH, dtype=F32), (8, 1))], axis=1)
    b0_1, b1_1 = jnp.tile(b1[:FH], 8).reshape(1, 128), jnp.tile(b1[FH:], 8).reshape(1, 128)
    b0_2, b1_2 = jnp.tile(b2[:FH], 8).reshape(1, 128), jnp.tile(b2[FH:], 8).reshape(1, 128)
    b0_3, b1_3 = jnp.tile(b3[:FH], 8).reshape(1, 128), jnp.tile(b3[FH:], 8).reshape(1, 128)
    attbr = att_b.reshape(1, 1)
    fcbr = fc_b.reshape(1, -1)

    x_p = jnp.pad(x, ((0, NP - N), (0, 0))).reshape(PK, 32)
    batch8 = jnp.pad(batch, (0, NP - N), constant_values=G).reshape(PK, 8)

    deg_flat = _deg_kernel(dst, w)
    degp = deg_flat.reshape(2, NPAD)
    d0 = degp[0, :NP].reshape(PK, 8)
    d1 = degp[1, :NP].reshape(PK, 8)

    dis, ga, gb = pl.pallas_call(
        _prep_body,
        grid=(NB8,),
        in_specs=[_pspec(8), _pspec(8), _pspec(32),
                  _full_spec((32, 128)), _full_spec((32, 128)),
                  _full_spec((8, 128))],
        out_specs=[_pspec(), _pspec(), _pspec()],
        out_shape=[jax.ShapeDtypeStruct((PK, 128), F32),
                   jax.ShapeDtypeStruct((PK, 128), F32),
                   jax.ShapeDtypeStruct((PK, 128), F32)],
    )(d0, d1, x_p, bw1a, bw1b, rmat)

    mid = pl.pallas_call(
        _mid_body,
        grid=(NB8,),
        in_specs=[_pspec(), _pspec(), _pspec(), _pspec(), _pspec(),
                  _full_spec((1, 128)), _full_spec((1, 128)),
                  _full_spec((128, 128)), _full_spec((128, 128)),
                  _full_spec((128, 128)), _full_spec((128, 128))],
        out_specs=[_pspec(), _pspec()],
        out_shape=[jax.ShapeDtypeStruct((PK, 128), F32),
                   jax.ShapeDtypeStruct((PK, 128), F32)],
    )

    def edge(ga_p, gb_p):
        oa, ob = _edge_kernel(src, dst, w,
                              ga_p.reshape(NP, FH), gb_p.reshape(NP, FH))
        return oa.reshape(PK, 128), ob.reshape(PK, 128)

    ma, mb = edge(ga, gb)
    ga, gb = mid(ma, mb, ga, gb, dis, b0_1, b1_1, waa2, wab2, wba2, wbb2)
    ma, mb = edge(ga, gb)
    ga, gb = mid(ma, mb, ga, gb, dis, b0_2, b1_2, waa3, wab3, wba3, wbb3)
    ma, mb = edge(ga, gb)

    h3a, h3b, att8, seg, cnt, stats = pl.pallas_call(
        _last_body,
        grid=(NB8,),
        in_specs=[_pspec(), _pspec(), _pspec(), _pspec(), _pspec(),
                  _full_spec((1, 128)), _full_spec((1, 128)),
                  _full_spec((128, 8)), _full_spec((128, 8)),
                  _full_spec((1, 1)), _pspec(8)],
        out_specs=[_pspec(), _pspec(), _pspec(8),
                   _full_spec((G, HID)), _full_spec((G, 1)),
                   pl.BlockSpec((1, 2), lambda i: (0, 0),
                                memory_space=pltpu.SMEM)],
        out_shape=[jax.ShapeDtypeStruct((PK, 128), F32),
                   jax.ShapeDtypeStruct((PK, 128), F32),
                   jax.ShapeDtypeStruct((PK, 8), F32),
                   jax.ShapeDtypeStruct((G, HID), F32),
                   jax.ShapeDtypeStruct((G, 1), F32),
                   jax.ShapeDtypeStruct((1, 2), F32)],
        scratch_shapes=[pltpu.SMEM((2,), F32)],
    )(ma, mb, ga, gb, dis, b0_3, b1_3, c0, c1, attbr, batch8)

    attn8, xwa, xwb = pl.pallas_call(
        _fin_body,
        grid=(NB8,),
        in_specs=[pl.BlockSpec((1, 2), lambda i: (0, 0),
                               memory_space=pltpu.SMEM),
                  _pspec(8), _pspec(), _pspec(), _full_spec((8, 128))],
        out_specs=[_pspec(8), _full_spec((1, 128)), _full_spec((1, 128))],
        out_shape=[jax.ShapeDtypeStruct((PK, 8), F32),
                   jax.ShapeDtypeStruct((1, 128), F32),
                   jax.ShapeDtypeStruct((1, 128), F32)],
    )(stats, att8, h3a, h3b, rmat)

    logw, xw = pl.pallas_call(
        _head_body,
        grid=(1,),
        in_specs=[_full_spec((G, HID)), _full_spec((G, 1)),
                  _full_spec(fc_W.shape), _full_spec((1, fc_W.shape[0])),
                  _full_spec((1, 128)), _full_spec((1, 128)),
                  _full_spec((128, HID)), _full_spec((128, HID))],
        out_specs=[_full_spec((G, fc_W.shape[0])), _full_spec((1, HID))],
        out_shape=[jax.ShapeDtypeStruct((G, fc_W.shape[0]), F32),
                   jax.ShapeDtypeStruct((1, HID), F32)],
    )(seg, cnt, fc_W, fcbr, xwa, xwb, s0, s1)

    attn = attn8.reshape(NP, 1)[:N]
    return (logw, attn, xw.reshape(HID))
